# Initial kernel scaffold; baseline (speedup 1.0000x reference)
import jax
import jax.numpy as jnp
from jax.experimental import pallas as pl


def kernel(x, codebook, steps):
    raise NotImplementedError("write your pallas kernel here")


# speedup vs baseline: 3.0509x; 3.0509x over previous
import functools

import jax
import jax.numpy as jnp
from jax.experimental import pallas as pl
from jax.experimental.pallas import tpu as pltpu


def _vq_block_kernel(x_ref, cb_ref, idx_ref, q_ref, counts_ref, diff_ref,
                     ent_ref, msum_ref, *, blk, n, c, bt, nblocks):
    step = pl.program_id(0)

    @pl.when(step == 0)
    def _init():
        counts_ref[...] = jnp.zeros_like(counts_ref)
        msum_ref[0] = 0.0

    x = x_ref[...]
    w = cb_ref[...]
    embed2 = jnp.sum(w * w, axis=1, keepdims=True)
    input2 = jnp.sum(x * x, axis=1, keepdims=True)
    xwt = jax.lax.dot_general(x, w, (((1,), (1,)), ((), ())),
                              preferred_element_type=jnp.float32)
    dist = (embed2.T + input2) - 2.0 * xwt
    min_d = jnp.min(dist, axis=1, keepdims=True)
    iota = jax.lax.broadcasted_iota(jnp.int32, (blk, n), 1)
    idx = jnp.min(jnp.where(dist == min_d, iota, n), axis=1,
                  keepdims=True)
    one_hot = (iota == idx).astype(jnp.float32)
    q = jax.lax.dot_general(one_hot, w, (((1,), (0,)), ((), ())),
                            preferred_element_type=jnp.float32)

    idx_ref[...] = idx
    q_ref[...] = q
    counts_ref[...] += jnp.sum(one_hot, axis=0, keepdims=True)
    msum_ref[0] += jnp.sum(min_d)

    @pl.when(step == nblocks - 1)
    def _finish():
        counts = counts_ref[...]
        p = counts / bt
        ent = jnp.exp(-jnp.sum(p * jnp.log(p + 1e-10)))
        ent_ref[...] = jnp.full((1, 1), ent, dtype=jnp.float32)
        diff_ref[...] = jnp.full((1, 1), msum_ref[0] / (bt * c),
                                 dtype=jnp.float32)


def kernel(x, codebook, steps):
    B, T, C = x.shape
    N = codebook.shape[0]
    BT = B * T
    BLK = 256
    nblocks = BT // BLK
    x_flat = x.reshape(BT, C)

    kfn = functools.partial(_vq_block_kernel, blk=BLK, n=N, c=C, bt=BT,
                            nblocks=nblocks)
    idx, q, counts, diff, ent = pl.pallas_call(
        kfn,
        grid=(nblocks,),
        in_specs=[
            pl.BlockSpec((BLK, C), lambda i: (i, 0)),
            pl.BlockSpec((N, C), lambda i: (0, 0)),
        ],
        out_specs=[
            pl.BlockSpec((BLK, 1), lambda i: (i, 0)),
            pl.BlockSpec((BLK, C), lambda i: (i, 0)),
            pl.BlockSpec((1, N), lambda i: (0, 0)),
            pl.BlockSpec((1, 1), lambda i: (0, 0)),
            pl.BlockSpec((1, 1), lambda i: (0, 0)),
        ],
        out_shape=[
            jax.ShapeDtypeStruct((BT, 1), jnp.int32),
            jax.ShapeDtypeStruct((BT, C), jnp.float32),
            jax.ShapeDtypeStruct((1, N), jnp.float32),
            jax.ShapeDtypeStruct((1, 1), jnp.float32),
            jax.ShapeDtypeStruct((1, 1), jnp.float32),
        ],
        scratch_shapes=[pltpu.SMEM((1,), jnp.float32)],
    )(x_flat, codebook)

    return idx, q.reshape(B, T, C), diff[0, 0], ent[0, 0]


# speedup vs baseline: 4.1521x; 1.3609x over previous
import functools

import jax
import jax.numpy as jnp
from jax.experimental import pallas as pl
from jax.experimental.pallas import tpu as pltpu


def _vq_block_kernel(x_ref, cb_ref, idx_ref, q_ref, counts_ref, diff_ref,
                     ent_ref, w2_ref, e2_ref, msum_ref,
                     *, blk, n, c, bt, nblocks):
    step = pl.program_id(0)

    @pl.when(step == 0)
    def _init():
        w = cb_ref[...]
        w2_ref[...] = -2.0 * w
        e2_ref[...] = jnp.sum(w * w, axis=1, keepdims=True).reshape(1, n)
        counts_ref[...] = jnp.zeros_like(counts_ref)
        msum_ref[0] = 0.0

    x = x_ref[...]
    xwt = jax.lax.dot_general(x, w2_ref[...], (((1,), (1,)), ((), ())),
                              preferred_element_type=jnp.float32)
    dist = xwt + e2_ref[...]
    min_d = jnp.min(dist, axis=1, keepdims=True)
    iota = jax.lax.broadcasted_iota(jnp.int32, (blk, n), 1)
    idx = jnp.min(jnp.where(dist == min_d, iota, n), axis=1,
                  keepdims=True)
    one_hot = (iota == idx).astype(jnp.float32)
    q = jax.lax.dot_general(one_hot, cb_ref[...], (((1,), (0,)), ((), ())),
                            preferred_element_type=jnp.float32)

    idx_ref[...] = idx
    q_ref[...] = q
    ones_row = jnp.ones((1, blk), dtype=jnp.float32)
    counts_ref[...] += jax.lax.dot_general(
        ones_row, one_hot, (((1,), (0,)), ((), ())),
        preferred_element_type=jnp.float32)
    msum_ref[0] += jnp.sum(min_d) + jnp.sum(x * x)

    @pl.when(step == nblocks - 1)
    def _finish():
        counts = counts_ref[...]
        p = counts / bt
        ent = jnp.exp(-jnp.sum(p * jnp.log(p + 1e-10)))
        ent_ref[...] = jnp.full((1, 1), ent, dtype=jnp.float32)
        diff_ref[...] = jnp.full((1, 1), msum_ref[0] / (bt * c),
                                 dtype=jnp.float32)


def kernel(x, codebook, steps):
    B, T, C = x.shape
    N = codebook.shape[0]
    BT = B * T
    BLK = 512
    nblocks = BT // BLK
    x_flat = x.reshape(BT, C)

    kfn = functools.partial(_vq_block_kernel, blk=BLK, n=N, c=C, bt=BT,
                            nblocks=nblocks)
    idx, q, counts, diff, ent = pl.pallas_call(
        kfn,
        grid=(nblocks,),
        in_specs=[
            pl.BlockSpec((BLK, C), lambda i: (i, 0)),
            pl.BlockSpec((N, C), lambda i: (0, 0)),
        ],
        out_specs=[
            pl.BlockSpec((BLK, 1), lambda i: (i, 0)),
            pl.BlockSpec((BLK, C), lambda i: (i, 0)),
            pl.BlockSpec((1, N), lambda i: (0, 0)),
            pl.BlockSpec((1, 1), lambda i: (0, 0)),
            pl.BlockSpec((1, 1), lambda i: (0, 0)),
        ],
        out_shape=[
            jax.ShapeDtypeStruct((BT, 1), jnp.int32),
            jax.ShapeDtypeStruct((BT, C), jnp.float32),
            jax.ShapeDtypeStruct((1, N), jnp.float32),
            jax.ShapeDtypeStruct((1, 1), jnp.float32),
            jax.ShapeDtypeStruct((1, 1), jnp.float32),
        ],
        scratch_shapes=[
            pltpu.VMEM((N, C), jnp.float32),
            pltpu.VMEM((1, N), jnp.float32),
            pltpu.SMEM((1,), jnp.float32),
        ],
    )(x_flat, codebook)

    return idx, q.reshape(B, T, C), diff[0, 0], ent[0, 0]
